# Initial kernel scaffold; baseline (speedup 1.0000x reference)
import jax
import jax.numpy as jnp
from jax.experimental import pallas as pl


def kernel(y, e, mu_causal, low_rank_causal, diag_causal, mu_spurious, low_rank_spurious, diag_spurious):
    raise NotImplementedError("write your pallas kernel here")


# speedup vs baseline: 2.2023x; 2.2023x over previous
import jax
import jax.numpy as jnp
from jax.experimental import pallas as pl
from jax.experimental.pallas import tpu as pltpu

Z = 128
R = 64
B = 1024
NCOMBO = 4
BB = 8


def _softplus(x):
    return jnp.maximum(x, 0.0) + jnp.log1p(jnp.exp(-jnp.abs(x)))


def _table_kernel(mu_c_ref, lr_c_ref, d_c_ref, mu_s_ref, lr_s_ref, d_s_ref,
                  cov_t_ref, mu_t_ref):
    row = jax.lax.broadcasted_iota(jnp.int32, (Z, Z), 0)
    col = jax.lax.broadcasted_iota(jnp.int32, (Z, Z), 1)
    diag_mask = (row == col).astype(jnp.float32)
    zeros_blk = jnp.zeros((Z, Z), dtype=jnp.float32)
    for combo in range(NCOMBO):
        e = combo % 2
        lrc = lr_c_ref[e]
        cc = jax.lax.dot_general(lrc, lrc, (((1,), (1,)), ((), ())),
                                 preferred_element_type=jnp.float32)
        dc = _softplus(d_c_ref[e]) + 1e-6
        cc = cc + diag_mask * dc[None, :]
        lrs = lr_s_ref[combo]
        cs = jax.lax.dot_general(lrs, lrs, (((1,), (1,)), ((), ())),
                                 preferred_element_type=jnp.float32)
        ds = _softplus(d_s_ref[combo]) + 1e-6
        cs = cs + diag_mask * ds[None, :]
        cov_t_ref[combo, 0:Z, 0:Z] = cc
        cov_t_ref[combo, 0:Z, Z:2 * Z] = zeros_blk
        cov_t_ref[combo, Z:2 * Z, 0:Z] = zeros_blk
        cov_t_ref[combo, Z:2 * Z, Z:2 * Z] = cs
        mu_t_ref[combo, 0:Z] = mu_c_ref[e, :]
        mu_t_ref[combo, Z:2 * Z] = mu_s_ref[combo, :]


def _build_tables(mu_causal, low_rank_causal, diag_causal,
                  mu_spurious, low_rank_spurious, diag_spurious):
    mu_s = mu_spurious.reshape(NCOMBO, Z)
    lr_s = low_rank_spurious.reshape(NCOMBO, Z, R)
    d_s = diag_spurious.reshape(NCOMBO, Z)
    return pl.pallas_call(
        _table_kernel,
        out_shape=(
            jax.ShapeDtypeStruct((NCOMBO, 2 * Z, 2 * Z), jnp.float32),
            jax.ShapeDtypeStruct((NCOMBO, 2 * Z), jnp.float32),
        ),
    )(mu_causal, low_rank_causal, diag_causal, mu_s, lr_s, d_s)


def _gather_kernel(combo_ref, cov_t_ref, mu_t_ref, cov_out_ref, mu_out_ref):
    i = pl.program_id(0)
    for j in range(BB):
        c = combo_ref[i * BB + j]
        cov_out_ref[j] = cov_t_ref[c]
        mu_out_ref[j] = mu_t_ref[c]


def kernel(y, e, mu_causal, low_rank_causal, diag_causal,
           mu_spurious, low_rank_spurious, diag_spurious):
    combo = (y.astype(jnp.int32) * 2 + e.astype(jnp.int32))
    cov_t, mu_t = _build_tables(mu_causal, low_rank_causal, diag_causal,
                                mu_spurious, low_rank_spurious, diag_spurious)
    b = y.shape[0]
    grid = (b // BB,)
    cov, mu = pl.pallas_call(
        _gather_kernel,
        grid_spec=pltpu.PrefetchScalarGridSpec(
            num_scalar_prefetch=1,
            grid=grid,
            in_specs=[
                pl.BlockSpec((NCOMBO, 2 * Z, 2 * Z), lambda i, c: (0, 0, 0)),
                pl.BlockSpec((NCOMBO, 2 * Z), lambda i, c: (0, 0)),
            ],
            out_specs=[
                pl.BlockSpec((BB, 2 * Z, 2 * Z), lambda i, c: (i, 0, 0)),
                pl.BlockSpec((BB, 2 * Z), lambda i, c: (i, 0)),
            ],
        ),
        out_shape=(
            jax.ShapeDtypeStruct((b, 2 * Z, 2 * Z), jnp.float32),
            jax.ShapeDtypeStruct((b, 2 * Z), jnp.float32),
        ),
    )(combo, cov_t, mu_t)
    return (mu, cov)
